# baseline (speedup 1.0000x reference)
import functools

import jax
import jax.numpy as jnp
from jax import lax
from jax.experimental import pallas as pl
from jax.experimental.pallas import tpu as pltpu
from jax.experimental.pallas import tpu_sc as plsc

VOCAB_SIZE = 100000
EMBED_DIM = 128
MAX_SEQ = 2048
BATCH = 4

_B = BATCH * MAX_SEQ
_INFO = plsc.get_sparse_core_info()
_NC = _INFO.num_cores
_NS = _INFO.num_subcores
_NW = _NC * _NS
_SPT = MAX_SEQ // _NW


@functools.partial(
    pl.kernel,
    mesh=plsc.VectorSubcoreMesh(core_axis_name="c", subcore_axis_name="s"),
    out_type=jax.ShapeDtypeStruct((_B, EMBED_DIM), jnp.float32),
    scratch_types=[
        pltpu.VMEM((BATCH * _SPT,), jnp.int32),
        pltpu.VMEM((BATCH * _SPT, EMBED_DIM), jnp.float32),
        pltpu.VMEM((_SPT, EMBED_DIM), jnp.float32),
        pltpu.SemaphoreType.DMA,
        pltpu.SemaphoreType.DMA,
        pltpu.SemaphoreType.DMA,
        pltpu.SemaphoreType.DMA,
        pltpu.SemaphoreType.DMA,
        pltpu.SemaphoreType.DMA,
    ],
)
def _sc_embed(idx_hbm, table_hbm, pos_hbm, out_hbm,
              idx_v, rows_v, pos_v, psem, wsem, gsem0, gsem1, gsem2, gsem3):
    wid = lax.axis_index("s") * _NC + lax.axis_index("c")
    col0 = wid * _SPT
    two = 2 * _SPT

    idx_cps = [
        pltpu.async_copy(idx_hbm.at[b, pl.ds(col0, _SPT)],
                         idx_v.at[pl.ds(b * _SPT, _SPT)], psem)
        for b in range(BATCH)
    ]
    for cp in idx_cps:
        cp.wait()

    pos_cp = pltpu.async_copy(pos_hbm.at[pl.ds(col0, _SPT)], pos_v, psem)
    gathers = []
    for b, gsem in enumerate((gsem0, gsem1, gsem2, gsem3)):
        gathers.append(pltpu.async_copy(
            table_hbm.at[idx_v.at[pl.ds(b * _SPT, _SPT)]],
            rows_v.at[pl.ds(b * _SPT, _SPT)],
            gsem))
    pos_cp.wait()

    writes = []
    for b in range(BATCH):
        gathers[b].wait()
        lo = b * _SPT

        def add_row(i, carry):
            for k in range(EMBED_DIM // 16):
                sl = pl.ds(k * 16, 16)
                plsc.addupdate(rows_v.at[lo + i, sl], pos_v[i, sl])
            return carry
        lax.fori_loop(0, _SPT, add_row, 0)
        writes.append(pltpu.async_copy(
            rows_v.at[pl.ds(lo, _SPT)],
            out_hbm.at[pl.ds(b * MAX_SEQ + col0, _SPT)],
            wsem))
    for cp in writes:
        cp.wait()


def kernel(tkn_ids, table, pos_encoding):
    idx = tkn_ids.astype(jnp.int32)
    pos = pos_encoding.reshape(MAX_SEQ, EMBED_DIM).astype(jnp.float32)
    out = _sc_embed(idx, table, pos)
    return out.reshape(BATCH, MAX_SEQ, EMBED_DIM)


# speedup vs baseline: 1.0104x; 1.0104x over previous
import functools

import jax
import jax.numpy as jnp
from jax import lax
from jax.experimental import pallas as pl
from jax.experimental.pallas import tpu as pltpu
from jax.experimental.pallas import tpu_sc as plsc

VOCAB_SIZE = 100000
EMBED_DIM = 128
MAX_SEQ = 2048
BATCH = 4

_B = BATCH * MAX_SEQ
_INFO = plsc.get_sparse_core_info()
_NC = _INFO.num_cores
_NS = _INFO.num_subcores
_NW = _NC * _NS
_SPT = MAX_SEQ // _NW


@functools.partial(
    pl.kernel,
    mesh=plsc.VectorSubcoreMesh(core_axis_name="c", subcore_axis_name="s"),
    out_type=jax.ShapeDtypeStruct((_B, EMBED_DIM), jnp.float32),
    scratch_types=[
        pltpu.VMEM((BATCH * _SPT,), jnp.int32),
        pltpu.VMEM((BATCH * _SPT, EMBED_DIM), jnp.float32),
        pltpu.VMEM((_SPT, EMBED_DIM), jnp.float32),
        pltpu.SemaphoreType.DMA,
        pltpu.SemaphoreType.DMA,
        pltpu.SemaphoreType.DMA,
        pltpu.SemaphoreType.DMA,
    ],
)
def _sc_embed(idx_hbm, table_hbm, pos_hbm, out_hbm,
              idx_v, rows_v, pos_v, psem, wsem, gsem0, gsem1):
    wid = lax.axis_index("s") * _NC + lax.axis_index("c")
    col0 = wid * _SPT
    two = 2 * _SPT

    idx_cps = [
        pltpu.async_copy(idx_hbm.at[b, pl.ds(col0, _SPT)],
                         idx_v.at[pl.ds(b * _SPT, _SPT)], psem)
        for b in range(BATCH)
    ]
    for cp in idx_cps:
        cp.wait()

    pos_cp = pltpu.async_copy(pos_hbm.at[pl.ds(col0, _SPT)], pos_v, psem)
    gathers = []
    for j, gsem in enumerate((gsem0, gsem1)):
        gathers.append(pltpu.async_copy(
            table_hbm.at[idx_v.at[pl.ds(j * two, two)]],
            rows_v.at[pl.ds(j * two, two)],
            gsem))
    pos_cp.wait()

    writes = []
    half = _SPT // 2
    for j in range(2):
        gathers[j].wait()
        lo = j * two

        def add_row(i, carry):
            for k in range(EMBED_DIM // 16):
                sl = pl.ds(k * 16, 16)
                p = pos_v[i, sl]
                plsc.addupdate(rows_v.at[lo + i, sl], p)
                plsc.addupdate(rows_v.at[lo + _SPT + i, sl], p)
            return carry
        lax.fori_loop(0, _SPT, add_row, 0)
        for h in range(2):
            b = 2 * j + h
            writes.append(pltpu.async_copy(
                rows_v.at[pl.ds(b * _SPT, _SPT)],
                out_hbm.at[pl.ds(b * MAX_SEQ + col0, _SPT)],
                wsem))
    for cp in writes:
        cp.wait()


def kernel(tkn_ids, table, pos_encoding):
    idx = tkn_ids.astype(jnp.int32)
    pos = pos_encoding.reshape(MAX_SEQ, EMBED_DIM).astype(jnp.float32)
    out = _sc_embed(idx, table, pos)
    return out.reshape(BATCH, MAX_SEQ, EMBED_DIM)
